# baseline (speedup 1.0000x reference)
import jax
import jax.numpy as jnp
from jax.experimental import pallas as pl
from jax.experimental.pallas import tpu as pltpu

D_H = 768
N_EXP = 8
N_M = 4
ND = N_M * D_H
D_F = int(D_H * 1.618)
TOP_P = 0.8
MAX_KSEL = 4
N_ACT = N_EXP - 1

TT = 256


def _dot_t(a, b, prec=None):
    return jax.lax.dot_general(
        a, b, (((1,), (1,)), ((), ())),
        preferred_element_type=jnp.float32, precision=prec)


def _moe_body(stream_ref, nw_ref, phi_ref, bias_ref, alpha_ref, swn_ref,
              wd_ref, wu_ref, wg_ref, wup_ref, wdn_ref, rw_ref,
              out_ref, gates_ref, lp_ref, gates_scr, rms_scr, amix_scr):
    e = pl.program_id(1)

    s0 = stream_ref[0]
    s1 = stream_ref[1]
    s2 = stream_ref[2]
    s3 = stream_ref[3]

    @pl.when(e == 0)
    def _router():
        xm = (s0 + s1 + s2 + s3) * 0.25
        logits = _dot_t(xm, rw_ref[...])
        m = jnp.max(logits, axis=1, keepdims=True)
        p = jnp.exp(logits - m)
        p = p / jnp.sum(p, axis=1, keepdims=True)
        colid = jax.lax.broadcasted_iota(jnp.int32, p.shape, 1)
        s_before = jnp.zeros_like(p)
        rank = jnp.zeros_like(p)
        for i in range(N_EXP):
            pi = p[:, i:i + 1]
            before = (pi > p) | ((pi == p) & (i < colid))
            bf = before.astype(jnp.float32)
            s_before = s_before + pi * bf
            rank = rank + bf
        mask = ((s_before < TOP_P) & (rank < MAX_KSEL)) | (rank == 0)
        gates = p * mask.astype(jnp.float32)
        gates_scr[...] = gates
        gates_ref[...] = gates
        logp = jnp.maximum(jnp.log(p), -10.0)
        lp_ref[...] = jnp.sum(
            logp * (gates > 0).astype(jnp.float32), axis=1, keepdims=True)
        ssq = (jnp.sum(s0 * s0, axis=1, keepdims=True)
               + jnp.sum(s1 * s1, axis=1, keepdims=True)
               + jnp.sum(s2 * s2, axis=1, keepdims=True)
               + jnp.sum(s3 * s3, axis=1, keepdims=True))
        rms_scr[...] = jax.lax.rsqrt(ssq * (1.0 / ND) + 1e-08)
        amix_scr[...] = jnp.zeros_like(amix_scr)
        out_ref[...] = jnp.zeros_like(out_ref)

    eidx = e + 1
    oh = (jax.lax.broadcasted_iota(jnp.int32, (1, N_EXP), 1) == eidx)
    gate_col = jnp.sum(
        gates_scr[...] * oh.astype(jnp.float32), axis=1, keepdims=True)

    @pl.when(jnp.max(gate_col) > 0.0)
    def _expert():
        streams = (s0, s1, s2, s3)
        rms = rms_scr[...]
        nwb = nw_ref[0]
        z = _dot_t(streams[0] * rms * nwb[0:1], phi_ref[0, 0])
        for n in range(1, N_M):
            z = z + _dot_t(streams[n] * rms * nwb[n:n + 1], phi_ref[0, n])
        a = alpha_ref[0]
        b = bias_ref[0]
        h_pre = jax.nn.sigmoid(z[:, 0:4] * a[:, 0:1] + b[:, 0:4])
        h_post = 2.0 * jax.nn.sigmoid(z[:, 4:8] * a[:, 1:2] + b[:, 4:8])
        mres = jnp.exp(z[:, 8:24] * a[:, 2:3] + b[:, 8:24])
        rid = jax.lax.broadcasted_iota(jnp.int32, (16, 16), 0)
        cid = jax.lax.broadcasted_iota(jnp.int32, (16, 16), 1)
        r_row = (rid // 4 == cid // 4).astype(jnp.float32)
        r_col = (rid % 4 == cid % 4).astype(jnp.float32)
        for _ in range(6):
            mres = mres / jnp.dot(mres, r_row,
                                  preferred_element_type=jnp.float32,
                                  precision=jax.lax.Precision.HIGHEST)
            mres = mres / jnp.dot(mres, r_col,
                                  preferred_element_type=jnp.float32,
                                  precision=jax.lax.Precision.HIGHEST)

        amix_scr[...] += gate_col * mres
        gpost = gate_col * h_post

        h_e = (h_pre[:, 0:1] * s0 + h_pre[:, 1:2] * s1
               + h_pre[:, 2:3] * s2 + h_pre[:, 3:4] * s3)
        ssq2 = jnp.sum(h_e * h_e, axis=1, keepdims=True)
        rms2 = jax.lax.rsqrt(ssq2 * (1.0 / D_H) + 1e-08)
        h = h_e * rms2 * swn_ref[0]

        hb = h.astype(jnp.bfloat16)
        wdo = _dot_t(hb, wd_ref[0])
        g = jax.nn.sigmoid(
            _dot_t(jax.nn.silu(wdo).astype(jnp.bfloat16), wu_ref[0]))
        go = _dot_t(hb, wg_ref[0])
        uo = _dot_t(hb, wup_ref[0])
        act = (jax.nn.silu(go) * uo).astype(jnp.bfloat16)
        out_e = g * _dot_t(act, wdn_ref[0])

        for n in range(N_M):
            out_ref[n] += gpost[:, n:n + 1] * out_e

    @pl.when(e == N_ACT - 1)
    def _finish():
        amix = amix_scr[...]
        for n in range(N_M):
            out_ref[n] += (amix[:, 4 * n:4 * n + 1] * s0
                           + amix[:, 4 * n + 1:4 * n + 2] * s1
                           + amix[:, 4 * n + 2:4 * n + 3] * s2
                           + amix[:, 4 * n + 3:4 * n + 4] * s3)


def kernel(stream, norm_w, phi_pre_w, phi_post_w, phi_res_w, b_pre, b_post,
           b_res, alpha_pre, alpha_post, alpha_res, swiglu_norm_w,
           swiglu_wd_w, swiglu_wu_w, swiglu_gate_w, swiglu_up_w,
           swiglu_down_w, router_w):
    Bs, n, T, d = stream.shape
    E = router_w.shape[0]
    s3 = stream[0]
    phi_cat = jnp.concatenate([phi_pre_w, phi_post_w, phi_res_w], axis=1)
    phi_cat = jnp.transpose(phi_cat.reshape(E, 24, N_M, d), (0, 2, 1, 3))
    bias_cat = jnp.concatenate(
        [b_pre, b_post, b_res.reshape(E, N_M * N_M)], axis=1)[:, None, :]
    alpha_cat = jnp.stack([alpha_pre, alpha_post, alpha_res], axis=1)[:, None, :]
    nw3 = norm_w.reshape(E, N_M, d)
    swn3 = swiglu_norm_w[:, None, :]
    wd_b = swiglu_wd_w.astype(jnp.bfloat16)
    wu_b = swiglu_wu_w.astype(jnp.bfloat16)
    wg_b = swiglu_gate_w.astype(jnp.bfloat16)
    wup_b = swiglu_up_w.astype(jnp.bfloat16)
    wdn_b = swiglu_down_w.astype(jnp.bfloat16)

    nt = T // TT
    grid = (nt, N_ACT)

    out, gates, lp = pl.pallas_call(
        _moe_body,
        grid=grid,
        in_specs=[
            pl.BlockSpec((N_M, TT, D_H), lambda tt, e: (0, tt, 0)),
            pl.BlockSpec((1, N_M, D_H), lambda tt, e: (e + 1, 0, 0)),
            pl.BlockSpec((1, N_M, 24, D_H), lambda tt, e: (e + 1, 0, 0, 0)),
            pl.BlockSpec((1, 1, 24), lambda tt, e: (e + 1, 0, 0)),
            pl.BlockSpec((1, 1, 3), lambda tt, e: (e + 1, 0, 0)),
            pl.BlockSpec((1, 1, D_H), lambda tt, e: (e + 1, 0, 0)),
            pl.BlockSpec((1, D_H, D_H), lambda tt, e: (e + 1, 0, 0)),
            pl.BlockSpec((1, D_H, D_H), lambda tt, e: (e + 1, 0, 0)),
            pl.BlockSpec((1, D_F, D_H), lambda tt, e: (e + 1, 0, 0)),
            pl.BlockSpec((1, D_F, D_H), lambda tt, e: (e + 1, 0, 0)),
            pl.BlockSpec((1, D_H, D_F), lambda tt, e: (e + 1, 0, 0)),
            pl.BlockSpec((N_EXP, D_H), lambda tt, e: (0, 0)),
        ],
        out_specs=[
            pl.BlockSpec((N_M, TT, D_H), lambda tt, e: (0, tt, 0)),
            pl.BlockSpec((TT, N_EXP), lambda tt, e: (tt, 0)),
            pl.BlockSpec((TT, 1), lambda tt, e: (tt, 0)),
        ],
        out_shape=[
            jax.ShapeDtypeStruct((N_M, T, D_H), jnp.float32),
            jax.ShapeDtypeStruct((T, N_EXP), jnp.float32),
            jax.ShapeDtypeStruct((T, 1), jnp.float32),
        ],
        scratch_shapes=[
            pltpu.VMEM((TT, N_EXP), jnp.float32),
            pltpu.VMEM((TT, 1), jnp.float32),
            pltpu.VMEM((TT, 16), jnp.float32),
        ],
        compiler_params=pltpu.CompilerParams(
            dimension_semantics=("arbitrary", "arbitrary"),
            vmem_limit_bytes=67_000_000,
        ),
    )(s3, nw3, phi_cat, bias_cat, alpha_cat, swn3,
      wd_b, wu_b, wg_b, wup_b, wdn_b,
      router_w)

    return out[None], gates[None], lp.reshape(1, T)


# speedup vs baseline: 1.6712x; 1.6712x over previous
import jax
import jax.numpy as jnp
from jax.experimental import pallas as pl
from jax.experimental.pallas import tpu as pltpu

D_H = 768
N_EXP = 8
N_M = 4
ND = N_M * D_H
D_F = int(D_H * 1.618)
TOP_P = 0.8
MAX_KSEL = 4
N_ACT = N_EXP - 1
NPP = 4 * N_ACT
NRR = 16 * N_ACT

TT = 256

_HI = jax.lax.Precision.HIGHEST


def _dot_t(a, b, prec=None):
    return jax.lax.dot_general(
        a, b, (((1,), (1,)), ((), ())),
        preferred_element_type=jnp.float32, precision=prec)


def _dot(a, b, prec=None):
    return jnp.dot(a, b, preferred_element_type=jnp.float32, precision=prec)


def _moe_body(stream_ref, phi_ref, ab_ref, swn_ref,
              wd_ref, wu_ref, wg_ref, wup_ref, wdn_ref, rw_ref,
              out_ref, gates_ref, lp_ref,
              gates_scr, hpre_scr, gpost_scr):
    e = pl.program_id(1)

    s0 = stream_ref[0]
    s1 = stream_ref[1]
    s2 = stream_ref[2]
    s3 = stream_ref[3]
    streams = (s0, s1, s2, s3)

    @pl.when(e == 0)
    def _per_tile():
        xm = (s0 + s1 + s2 + s3) * 0.25
        logits = _dot_t(xm, rw_ref[...])
        m = jnp.max(logits, axis=1, keepdims=True)
        p = jnp.exp(logits - m)
        p = p / jnp.sum(p, axis=1, keepdims=True)
        colid = jax.lax.broadcasted_iota(jnp.int32, p.shape, 1)
        s_before = jnp.zeros_like(p)
        rank = jnp.zeros_like(p)
        for i in range(N_EXP):
            pi = p[:, i:i + 1]
            before = (pi > p) | ((pi == p) & (i < colid))
            bf = before.astype(jnp.float32)
            s_before = s_before + pi * bf
            rank = rank + bf
        mask = ((s_before < TOP_P) & (rank < MAX_KSEL)) | (rank == 0)
        gates = p * mask.astype(jnp.float32)
        gates_scr[...] = gates
        gates_ref[...] = gates
        logp = jnp.maximum(jnp.log(p), -10.0)
        lp_ref[...] = jnp.sum(
            logp * (gates > 0).astype(jnp.float32), axis=1, keepdims=True)

        ssq = (jnp.sum(s0 * s0, axis=1, keepdims=True)
               + jnp.sum(s1 * s1, axis=1, keepdims=True)
               + jnp.sum(s2 * s2, axis=1, keepdims=True)
               + jnp.sum(s3 * s3, axis=1, keepdims=True))
        rms = jax.lax.rsqrt(ssq * (1.0 / ND) + 1e-08)

        z = _dot(streams[0] * rms, phi_ref[0])
        for n in range(1, N_M):
            z = z + _dot(streams[n] * rms, phi_ref[n])
        ab = ab_ref[...]
        apre, bpre = ab[:, 0:NPP], ab[:, NPP:2 * NPP]
        apost, bpost = ab[:, 56:56 + NPP], ab[:, 84:84 + NPP]
        ares, bres = ab[:, 112:112 + NRR], ab[:, 224:224 + NRR]

        hpre_all = jax.nn.sigmoid(z[:, 0:NPP] * apre + bpre)
        u8 = jax.lax.broadcasted_iota(jnp.int32, (N_EXP, NPP), 0)
        q28 = jax.lax.broadcasted_iota(jnp.int32, (N_EXP, NPP), 1)
        g28m = (u8 == q28 // 4 + 1).astype(jnp.float32)
        gate28 = _dot(gates, g28m, _HI)
        gpost_all = gate28 * (
            2.0 * jax.nn.sigmoid(z[:, NPP:2 * NPP] * apost + bpost))

        mres = jnp.exp(z[:, 2 * NPP:] * ares + bres)
        rid = jax.lax.broadcasted_iota(jnp.int32, (NRR, NRR), 0)
        cid = jax.lax.broadcasted_iota(jnp.int32, (NRR, NRR), 1)
        r_row = (rid // 4 == cid // 4).astype(jnp.float32)
        r_col = ((rid // 16 == cid // 16)
                 & (rid % 4 == cid % 4)).astype(jnp.float32)
        for _ in range(6):
            mres = mres / _dot(mres, r_row, _HI)
            mres = mres / _dot(mres, r_col, _HI)

        u8r = jax.lax.broadcasted_iota(jnp.int32, (N_EXP, NRR), 0)
        q112 = jax.lax.broadcasted_iota(jnp.int32, (N_EXP, NRR), 1)
        g112m = (u8r == q112 // 16 + 1).astype(jnp.float32)
        gate112 = _dot(gates, g112m, _HI)
        p112 = jax.lax.broadcasted_iota(jnp.int32, (NRR, 16), 0)
        m16 = jax.lax.broadcasted_iota(jnp.int32, (NRR, 16), 1)
        s112 = (p112 % 16 == m16).astype(jnp.float32)
        amix = _dot(mres * gate112, s112, _HI)

        for n in range(N_M):
            out_ref[n] = (amix[:, 4 * n:4 * n + 1] * s0
                          + amix[:, 4 * n + 1:4 * n + 2] * s1
                          + amix[:, 4 * n + 2:4 * n + 3] * s2
                          + amix[:, 4 * n + 3:4 * n + 4] * s3)

        for k in range(N_ACT):
            hpre_scr[k] = hpre_all[:, 4 * k:4 * k + 4]
            gpost_scr[k] = gpost_all[:, 4 * k:4 * k + 4]

    eidx = e + 1
    oh = (jax.lax.broadcasted_iota(jnp.int32, (1, N_EXP), 1) == eidx)
    gate_col = jnp.sum(
        gates_scr[...] * oh.astype(jnp.float32), axis=1, keepdims=True)

    @pl.when(jnp.max(gate_col) > 0.0)
    def _expert():
        hp = hpre_scr[e]
        h_e = (hp[:, 0:1] * s0 + hp[:, 1:2] * s1
               + hp[:, 2:3] * s2 + hp[:, 3:4] * s3)
        ssq2 = jnp.sum(h_e * h_e, axis=1, keepdims=True)
        rms2 = jax.lax.rsqrt(ssq2 * (1.0 / D_H) + 1e-08)
        h = h_e * rms2 * swn_ref[0]

        hb = h.astype(jnp.bfloat16)
        wdo = _dot_t(hb, wd_ref[0])
        g = jax.nn.sigmoid(
            _dot_t(jax.nn.silu(wdo).astype(jnp.bfloat16), wu_ref[0]))
        go = _dot_t(hb, wg_ref[0])
        uo = _dot_t(hb, wup_ref[0])
        act = (jax.nn.silu(go) * uo).astype(jnp.bfloat16)
        out_e = g * _dot_t(act, wdn_ref[0])

        gp = gpost_scr[e]
        for n in range(N_M):
            out_ref[n] += gp[:, n:n + 1] * out_e


def kernel(stream, norm_w, phi_pre_w, phi_post_w, phi_res_w, b_pre, b_post,
           b_res, alpha_pre, alpha_post, alpha_res, swiglu_norm_w,
           swiglu_wd_w, swiglu_wu_w, swiglu_gate_w, swiglu_up_w,
           swiglu_down_w, router_w):
    Bs, n, T, d = stream.shape
    E = router_w.shape[0]
    s3 = stream[0]

    nw = norm_w.reshape(E, 1, N_M, d)
    pre = (phi_pre_w.reshape(E, 4, N_M, d) * nw)[1:]
    post = (phi_post_w.reshape(E, 4, N_M, d) * nw)[1:]
    res = (phi_res_w.reshape(E, 16, N_M, d) * nw)[1:]
    pre_m = jnp.transpose(pre, (2, 3, 0, 1)).reshape(N_M, d, NPP)
    post_m = jnp.transpose(post, (2, 3, 0, 1)).reshape(N_M, d, NPP)
    res_m = jnp.transpose(res, (2, 3, 0, 1)).reshape(N_M, d, NRR)
    phi_mat = jnp.concatenate([pre_m, post_m, res_m], axis=2)

    ab = jnp.concatenate([
        jnp.repeat(alpha_pre[1:], 4), b_pre[1:].reshape(-1),
        jnp.repeat(alpha_post[1:], 4), b_post[1:].reshape(-1),
        jnp.repeat(alpha_res[1:], 16), b_res[1:].reshape(-1),
    ])[None, :]

    swn3 = swiglu_norm_w[:, None, :]
    wd_b = swiglu_wd_w.astype(jnp.bfloat16)
    wu_b = swiglu_wu_w.astype(jnp.bfloat16)
    wg_b = swiglu_gate_w.astype(jnp.bfloat16)
    wup_b = swiglu_up_w.astype(jnp.bfloat16)
    wdn_b = swiglu_down_w.astype(jnp.bfloat16)

    nt = T // TT
    grid = (nt, N_ACT)

    out, gates, lp = pl.pallas_call(
        _moe_body,
        grid=grid,
        in_specs=[
            pl.BlockSpec((N_M, TT, D_H), lambda tt, e: (0, tt, 0)),
            pl.BlockSpec((N_M, D_H, NPP + NPP + NRR),
                         lambda tt, e: (0, 0, 0)),
            pl.BlockSpec((1, 336), lambda tt, e: (0, 0)),
            pl.BlockSpec((1, 1, D_H), lambda tt, e: (e + 1, 0, 0)),
            pl.BlockSpec((1, D_H, D_H), lambda tt, e: (e + 1, 0, 0)),
            pl.BlockSpec((1, D_H, D_H), lambda tt, e: (e + 1, 0, 0)),
            pl.BlockSpec((1, D_F, D_H), lambda tt, e: (e + 1, 0, 0)),
            pl.BlockSpec((1, D_F, D_H), lambda tt, e: (e + 1, 0, 0)),
            pl.BlockSpec((1, D_H, D_F), lambda tt, e: (e + 1, 0, 0)),
            pl.BlockSpec((N_EXP, D_H), lambda tt, e: (0, 0)),
        ],
        out_specs=[
            pl.BlockSpec((N_M, TT, D_H), lambda tt, e: (0, tt, 0)),
            pl.BlockSpec((TT, N_EXP), lambda tt, e: (tt, 0)),
            pl.BlockSpec((TT, 1), lambda tt, e: (tt, 0)),
        ],
        out_shape=[
            jax.ShapeDtypeStruct((N_M, T, D_H), jnp.float32),
            jax.ShapeDtypeStruct((T, N_EXP), jnp.float32),
            jax.ShapeDtypeStruct((T, 1), jnp.float32),
        ],
        scratch_shapes=[
            pltpu.VMEM((TT, N_EXP), jnp.float32),
            pltpu.VMEM((N_ACT, TT, 4), jnp.float32),
            pltpu.VMEM((N_ACT, TT, 4), jnp.float32),
        ],
        compiler_params=pltpu.CompilerParams(
            dimension_semantics=("arbitrary", "arbitrary"),
            vmem_limit_bytes=67_000_000,
        ),
    )(s3, phi_mat, ab, swn3,
      wd_b, wu_b, wg_b, wup_b, wdn_b,
      router_w)

    return out[None], gates[None], lp.reshape(1, T)


# speedup vs baseline: 1.8046x; 1.0798x over previous
import jax
import jax.numpy as jnp
from jax.experimental import pallas as pl
from jax.experimental.pallas import tpu as pltpu

D_H = 768
N_EXP = 8
N_M = 4
ND = N_M * D_H
D_F = int(D_H * 1.618)
TOP_P = 0.8
MAX_KSEL = 4
N_ACT = N_EXP - 1
NPP = 4 * N_ACT
NRR = 16 * N_ACT

TT = 512

_HI = jax.lax.Precision.HIGHEST


def _dot_t(a, b, prec=None):
    return jax.lax.dot_general(
        a, b, (((1,), (1,)), ((), ())),
        preferred_element_type=jnp.float32, precision=prec)


def _dot(a, b, prec=None):
    return jnp.dot(a, b, preferred_element_type=jnp.float32, precision=prec)


def _moe_body(stream_ref, phi_ref, ab_ref, swn_ref,
              wd_ref, wu_ref, wg_ref, wup_ref, wdn_ref, rw_ref,
              out_ref, gates_ref, lp_ref,
              gates_scr, hpre_scr, gpost_scr):
    e = pl.program_id(1)

    s0 = stream_ref[0]
    s1 = stream_ref[1]
    s2 = stream_ref[2]
    s3 = stream_ref[3]
    streams = (s0, s1, s2, s3)

    @pl.when(e == 0)
    def _per_tile():
        xm = (s0 + s1 + s2 + s3) * 0.25
        logits = _dot_t(xm, rw_ref[...])
        m = jnp.max(logits, axis=1, keepdims=True)
        p = jnp.exp(logits - m)
        p = p / jnp.sum(p, axis=1, keepdims=True)
        colid = jax.lax.broadcasted_iota(jnp.int32, p.shape, 1)
        s_before = jnp.zeros_like(p)
        rank = jnp.zeros_like(p)
        for i in range(N_EXP):
            pi = p[:, i:i + 1]
            before = (pi > p) | ((pi == p) & (i < colid))
            bf = before.astype(jnp.float32)
            s_before = s_before + pi * bf
            rank = rank + bf
        mask = ((s_before < TOP_P) & (rank < MAX_KSEL)) | (rank == 0)
        gates = p * mask.astype(jnp.float32)
        gates_scr[...] = gates
        gates_ref[...] = gates
        logp = jnp.maximum(jnp.log(p), -10.0)
        lp_ref[...] = jnp.sum(
            logp * (gates > 0).astype(jnp.float32), axis=1, keepdims=True)

        ssq = (jnp.sum(s0 * s0, axis=1, keepdims=True)
               + jnp.sum(s1 * s1, axis=1, keepdims=True)
               + jnp.sum(s2 * s2, axis=1, keepdims=True)
               + jnp.sum(s3 * s3, axis=1, keepdims=True))
        rms = jax.lax.rsqrt(ssq * (1.0 / ND) + 1e-08)

        z = _dot(streams[0] * rms, phi_ref[0])
        for n in range(1, N_M):
            z = z + _dot(streams[n] * rms, phi_ref[n])
        ab = ab_ref[...]
        apre, bpre = ab[:, 0:NPP], ab[:, NPP:2 * NPP]
        apost, bpost = ab[:, 56:56 + NPP], ab[:, 84:84 + NPP]
        ares, bres = ab[:, 112:112 + NRR], ab[:, 224:224 + NRR]

        hpre_all = jax.nn.sigmoid(z[:, 0:NPP] * apre + bpre)
        u8 = jax.lax.broadcasted_iota(jnp.int32, (N_EXP, NPP), 0)
        q28 = jax.lax.broadcasted_iota(jnp.int32, (N_EXP, NPP), 1)
        g28m = (u8 == q28 // 4 + 1).astype(jnp.float32)
        gate28 = _dot(gates, g28m, _HI)
        gpost_all = gate28 * (
            2.0 * jax.nn.sigmoid(z[:, NPP:2 * NPP] * apost + bpost))

        mres = jnp.exp(z[:, 2 * NPP:] * ares + bres)
        rid = jax.lax.broadcasted_iota(jnp.int32, (NRR, NRR), 0)
        cid = jax.lax.broadcasted_iota(jnp.int32, (NRR, NRR), 1)
        r_row = (rid // 4 == cid // 4).astype(jnp.float32)
        r_col = ((rid // 16 == cid // 16)
                 & (rid % 4 == cid % 4)).astype(jnp.float32)
        for _ in range(6):
            mres = mres / _dot(mres, r_row, _HI)
            mres = mres / _dot(mres, r_col, _HI)

        u8r = jax.lax.broadcasted_iota(jnp.int32, (N_EXP, NRR), 0)
        q112 = jax.lax.broadcasted_iota(jnp.int32, (N_EXP, NRR), 1)
        g112m = (u8r == q112 // 16 + 1).astype(jnp.float32)
        gate112 = _dot(gates, g112m, _HI)
        p112 = jax.lax.broadcasted_iota(jnp.int32, (NRR, 16), 0)
        m16 = jax.lax.broadcasted_iota(jnp.int32, (NRR, 16), 1)
        s112 = (p112 % 16 == m16).astype(jnp.float32)
        amix = _dot(mres * gate112, s112, _HI)

        for n in range(N_M):
            out_ref[n] = (amix[:, 4 * n:4 * n + 1] * s0
                          + amix[:, 4 * n + 1:4 * n + 2] * s1
                          + amix[:, 4 * n + 2:4 * n + 3] * s2
                          + amix[:, 4 * n + 3:4 * n + 4] * s3)

        for k in range(N_ACT):
            hpre_scr[k] = hpre_all[:, 4 * k:4 * k + 4]
            gpost_scr[k] = gpost_all[:, 4 * k:4 * k + 4]

    eidx = e + 1
    oh = (jax.lax.broadcasted_iota(jnp.int32, (1, N_EXP), 1) == eidx)
    gate_col = jnp.sum(
        gates_scr[...] * oh.astype(jnp.float32), axis=1, keepdims=True)

    @pl.when(jnp.max(gate_col) > 0.0)
    def _expert():
        hp = hpre_scr[e]
        h_e = (hp[:, 0:1] * s0 + hp[:, 1:2] * s1
               + hp[:, 2:3] * s2 + hp[:, 3:4] * s3)
        ssq2 = jnp.sum(h_e * h_e, axis=1, keepdims=True)
        rms2 = jax.lax.rsqrt(ssq2 * (1.0 / D_H) + 1e-08)
        h = h_e * rms2 * swn_ref[0]

        hb = h.astype(jnp.bfloat16)
        wdo = _dot_t(hb, wd_ref[0])
        g = jax.nn.sigmoid(
            _dot_t(jax.nn.silu(wdo).astype(jnp.bfloat16), wu_ref[0]))
        go = _dot_t(hb, wg_ref[0])
        uo = _dot_t(hb, wup_ref[0])
        act = (jax.nn.silu(go) * uo).astype(jnp.bfloat16)
        out_e = g * _dot_t(act, wdn_ref[0])

        gp = gpost_scr[e]
        for n in range(N_M):
            out_ref[n] += gp[:, n:n + 1] * out_e


def kernel(stream, norm_w, phi_pre_w, phi_post_w, phi_res_w, b_pre, b_post,
           b_res, alpha_pre, alpha_post, alpha_res, swiglu_norm_w,
           swiglu_wd_w, swiglu_wu_w, swiglu_gate_w, swiglu_up_w,
           swiglu_down_w, router_w):
    Bs, n, T, d = stream.shape
    E = router_w.shape[0]
    s3 = stream[0]

    nw = norm_w.reshape(E, 1, N_M, d)
    pre = (phi_pre_w.reshape(E, 4, N_M, d) * nw)[1:]
    post = (phi_post_w.reshape(E, 4, N_M, d) * nw)[1:]
    res = (phi_res_w.reshape(E, 16, N_M, d) * nw)[1:]
    pre_m = jnp.transpose(pre, (2, 3, 0, 1)).reshape(N_M, d, NPP)
    post_m = jnp.transpose(post, (2, 3, 0, 1)).reshape(N_M, d, NPP)
    res_m = jnp.transpose(res, (2, 3, 0, 1)).reshape(N_M, d, NRR)
    phi_mat = jnp.concatenate([pre_m, post_m, res_m], axis=2)

    ab = jnp.concatenate([
        jnp.repeat(alpha_pre[1:], 4), b_pre[1:].reshape(-1),
        jnp.repeat(alpha_post[1:], 4), b_post[1:].reshape(-1),
        jnp.repeat(alpha_res[1:], 16), b_res[1:].reshape(-1),
    ])[None, :]

    swn3 = swiglu_norm_w[:, None, :]
    wd_b = swiglu_wd_w.astype(jnp.bfloat16)
    wu_b = swiglu_wu_w.astype(jnp.bfloat16)
    wg_b = swiglu_gate_w.astype(jnp.bfloat16)
    wup_b = swiglu_up_w.astype(jnp.bfloat16)
    wdn_b = swiglu_down_w.astype(jnp.bfloat16)

    nt = T // TT
    grid = (nt, N_ACT)

    out, gates, lp = pl.pallas_call(
        _moe_body,
        grid=grid,
        in_specs=[
            pl.BlockSpec((N_M, TT, D_H), lambda tt, e: (0, tt, 0)),
            pl.BlockSpec((N_M, D_H, NPP + NPP + NRR),
                         lambda tt, e: (0, 0, 0)),
            pl.BlockSpec((1, 336), lambda tt, e: (0, 0)),
            pl.BlockSpec((1, 1, D_H), lambda tt, e: (e + 1, 0, 0)),
            pl.BlockSpec((1, D_H, D_H), lambda tt, e: (e + 1, 0, 0)),
            pl.BlockSpec((1, D_H, D_H), lambda tt, e: (e + 1, 0, 0)),
            pl.BlockSpec((1, D_F, D_H), lambda tt, e: (e + 1, 0, 0)),
            pl.BlockSpec((1, D_F, D_H), lambda tt, e: (e + 1, 0, 0)),
            pl.BlockSpec((1, D_H, D_F), lambda tt, e: (e + 1, 0, 0)),
            pl.BlockSpec((N_EXP, D_H), lambda tt, e: (0, 0)),
        ],
        out_specs=[
            pl.BlockSpec((N_M, TT, D_H), lambda tt, e: (0, tt, 0)),
            pl.BlockSpec((TT, N_EXP), lambda tt, e: (tt, 0)),
            pl.BlockSpec((TT, 1), lambda tt, e: (tt, 0)),
        ],
        out_shape=[
            jax.ShapeDtypeStruct((N_M, T, D_H), jnp.float32),
            jax.ShapeDtypeStruct((T, N_EXP), jnp.float32),
            jax.ShapeDtypeStruct((T, 1), jnp.float32),
        ],
        scratch_shapes=[
            pltpu.VMEM((TT, N_EXP), jnp.float32),
            pltpu.VMEM((N_ACT, TT, 4), jnp.float32),
            pltpu.VMEM((N_ACT, TT, 4), jnp.float32),
        ],
        compiler_params=pltpu.CompilerParams(
            dimension_semantics=("arbitrary", "arbitrary"),
            vmem_limit_bytes=67_000_000,
        ),
    )(s3, phi_mat, ab, swn3,
      wd_b, wu_b, wg_b, wup_b, wdn_b,
      router_w)

    return out[None], gates[None], lp.reshape(1, T)
